# baseline (speedup 1.0000x reference)
import jax
import jax.numpy as jnp
from jax.experimental import pallas as pl
from jax.experimental.pallas import tpu as pltpu

CBH = 1
NBUF = 16
DEPTH = 8


def _body(nt_ref, k_ref, kc_ref, v_ref, vc_ref, ok_ref, ov_ref,
          bufs, krows, vrows, sem_rows, sems_in, sems_out):
    BH, S, D = kc_ref.shape
    n_chunks = 2 * (BH // CBH)
    nt = nt_ref[0]

    ld_k = pltpu.make_async_copy(k_ref, krows, sem_rows)
    ld_v = pltpu.make_async_copy(v_ref, vrows, sem_rows)
    ld_k.start()
    ld_v.start()

    def chunk_refs(i):
        cache, bh = i % 2, (i // 2) * CBH
        src = kc_ref if cache == 0 else vc_ref
        dst = ok_ref if cache == 0 else ov_ref
        rows = krows if cache == 0 else vrows
        return src.at[pl.ds(bh, CBH)], dst.at[pl.ds(bh, CBH)], rows, bh

    def start_in(i):
        src, _, _, _ = chunk_refs(i)
        cp = pltpu.make_async_copy(src, bufs.at[i % NBUF], sems_in.at[i % NBUF])
        cp.start()
        return cp

    def drain(i, in_copies):
        _, dst, rows, bh = chunk_refs(i)
        slot = i % NBUF
        if i == 0:
            ld_k.wait()
            ld_v.wait()
        in_copies[i].wait()
        for c in range(CBH):
            bufs[slot, c, pl.ds(nt, 1), :] = rows[bh + c]
        cp = pltpu.make_async_copy(bufs.at[slot], dst, sems_out.at[slot])
        cp.start()
        return cp

    in_copies, out_copies = {}, {}
    for i in range(n_chunks):
        if i >= NBUF:
            out_copies[i - NBUF].wait()
        in_copies[i] = start_in(i)
        if i >= DEPTH:
            out_copies[i - DEPTH] = drain(i - DEPTH, in_copies)
    for j in range(n_chunks - DEPTH, n_chunks):
        out_copies[j] = drain(j, in_copies)
    for j in range(max(0, n_chunks - NBUF), n_chunks):
        out_copies[j].wait()


def kernel(k, k_cache, v, v_cache, n_tokens):
    B, H, S, D = k_cache.shape
    BH = B * H
    nt = jnp.asarray(n_tokens, jnp.int32).reshape(1)
    k2 = k.reshape(BH, 1, D)
    v2 = v.reshape(BH, 1, D)
    kc = k_cache.reshape(BH, S, D)
    vc = v_cache.reshape(BH, S, D)

    any_spec = pl.BlockSpec(memory_space=pl.ANY)
    out_k, out_v = pl.pallas_call(
        _body,
        in_specs=[
            pl.BlockSpec(memory_space=pltpu.SMEM),
            any_spec, any_spec, any_spec, any_spec,
        ],
        out_specs=[any_spec, any_spec],
        out_shape=[
            jax.ShapeDtypeStruct((BH, S, D), k_cache.dtype),
            jax.ShapeDtypeStruct((BH, S, D), v_cache.dtype),
        ],
        scratch_shapes=(
            [pltpu.VMEM((NBUF, CBH, S, D), k_cache.dtype),
             pltpu.VMEM((BH, 1, D), k.dtype),
             pltpu.VMEM((BH, 1, D), v.dtype),
             pltpu.SemaphoreType.DMA,
             pltpu.SemaphoreType.DMA((NBUF,)),
             pltpu.SemaphoreType.DMA((NBUF,))]
        ),
    )(nt, k2, kc, v2, vc)
    return (out_k.reshape(B, H, S, D), out_v.reshape(B, H, S, D))


# speedup vs baseline: 1.0064x; 1.0064x over previous
import jax
import jax.numpy as jnp
from jax.experimental import pallas as pl
from jax.experimental.pallas import tpu as pltpu

CBH = 2
NBUF = 12
DEPTH = 6


def _body(nt_ref, k_ref, kc_ref, v_ref, vc_ref, ok_ref, ov_ref,
          bufs, krows, vrows, sem_rows, sems_in, sems_out):
    BH, S, D = kc_ref.shape
    n_chunks = 2 * (BH // CBH)
    nt = nt_ref[0]

    ld_k = pltpu.make_async_copy(k_ref, krows, sem_rows)
    ld_v = pltpu.make_async_copy(v_ref, vrows, sem_rows)
    ld_k.start()
    ld_v.start()

    def chunk_refs(i):
        cache, bh = i % 2, (i // 2) * CBH
        src = kc_ref if cache == 0 else vc_ref
        dst = ok_ref if cache == 0 else ov_ref
        rows = krows if cache == 0 else vrows
        return src.at[pl.ds(bh, CBH)], dst.at[pl.ds(bh, CBH)], rows, bh

    def start_in(i):
        src, _, _, _ = chunk_refs(i)
        cp = pltpu.make_async_copy(src, bufs.at[i % NBUF], sems_in.at[i % NBUF])
        cp.start()
        return cp

    def drain(i, in_copies):
        _, dst, rows, bh = chunk_refs(i)
        slot = i % NBUF
        if i == 0:
            ld_k.wait()
            ld_v.wait()
        in_copies[i].wait()
        for c in range(CBH):
            bufs[slot, c, pl.ds(nt, 1), :] = rows[bh + c]
        cp = pltpu.make_async_copy(bufs.at[slot], dst, sems_out.at[slot])
        cp.start()
        return cp

    in_copies, out_copies = {}, {}
    for i in range(n_chunks):
        if i >= NBUF:
            out_copies[i - NBUF].wait()
        in_copies[i] = start_in(i)
        if i >= DEPTH:
            out_copies[i - DEPTH] = drain(i - DEPTH, in_copies)
    for j in range(n_chunks - DEPTH, n_chunks):
        out_copies[j] = drain(j, in_copies)
    for j in range(max(0, n_chunks - NBUF), n_chunks):
        out_copies[j].wait()


def kernel(k, k_cache, v, v_cache, n_tokens):
    B, H, S, D = k_cache.shape
    BH = B * H
    nt = jnp.asarray(n_tokens, jnp.int32).reshape(1)
    k2 = k.reshape(BH, 1, D)
    v2 = v.reshape(BH, 1, D)
    kc = k_cache.reshape(BH, S, D)
    vc = v_cache.reshape(BH, S, D)

    any_spec = pl.BlockSpec(memory_space=pl.ANY)
    out_k, out_v = pl.pallas_call(
        _body,
        in_specs=[
            pl.BlockSpec(memory_space=pltpu.SMEM),
            any_spec, any_spec, any_spec, any_spec,
        ],
        out_specs=[any_spec, any_spec],
        out_shape=[
            jax.ShapeDtypeStruct((BH, S, D), k_cache.dtype),
            jax.ShapeDtypeStruct((BH, S, D), v_cache.dtype),
        ],
        scratch_shapes=(
            [pltpu.VMEM((NBUF, CBH, S, D), k_cache.dtype),
             pltpu.VMEM((BH, 1, D), k.dtype),
             pltpu.VMEM((BH, 1, D), v.dtype),
             pltpu.SemaphoreType.DMA,
             pltpu.SemaphoreType.DMA((NBUF,)),
             pltpu.SemaphoreType.DMA((NBUF,))]
        ),
    )(nt, k2, kc, v2, vc)
    return (out_k.reshape(B, H, S, D), out_v.reshape(B, H, S, D))
